# baseline (speedup 1.0000x reference)
import functools

import jax
import jax.numpy as jnp
from jax import lax
from jax.experimental import pallas as pl
from jax.experimental.pallas import tpu as pltpu
from jax.experimental.pallas import tpu_sc as plsc

N = 10000
D = 128
E = 320000

NC = 2
NS = 16
NW = NC * NS

K = 80
CPW = 125
EPW = CPW * K
NP = 10000
RPT = 632
RPT_LAST = NP - 15 * RPT

_mesh = lambda: plsc.VectorSubcoreMesh(core_axis_name="c", subcore_axis_name="s")


def _deg_body(col_hbm, out_hbm, cidx, hist):
    c = lax.axis_index("c")
    s = lax.axis_index("s")
    wid = s * NC + c
    pltpu.sync_copy(col_hbm.at[pl.ds(wid * EPW, EPW)], cidx)
    zeros16 = jnp.zeros((16,), jnp.float32)

    def zstep(i, _):
        hist[pl.ds(i * 16, 16)] = zeros16
        return _

    lax.fori_loop(0, NP // 16, zstep, None)
    ones16 = jnp.ones((16,), jnp.float32)

    def step(j, _):
        ids = cidx[pl.ds(j * 16, 16)]
        plsc.addupdate_scatter(hist, [ids], ones16)
        return _

    lax.fori_loop(0, EPW // 16, step, None)
    pltpu.sync_copy(hist, out_hbm.at[pl.ds(wid * NP, NP)])


_deg_kernel = functools.partial(
    pl.kernel,
    out_type=jax.ShapeDtypeStruct((NW * NP,), jnp.float32),
    mesh=_mesh(),
    scratch_types=[
        pltpu.VMEM((EPW,), jnp.int32),
        pltpu.VMEM((NP,), jnp.float32),
    ],
    compiler_params=pltpu.CompilerParams(needs_layout_passes=False),
)(_deg_body)


def _edge_body(hs_hbm, row_hbm, col_hbm, zeros_hbm, out_hbm,
               ridx, cidx, rows0, rows1, rows2, acc_sh,
               gs0, gs1, gs2, ss0, ss1, ss2):
    c = lax.axis_index("c")
    s = lax.axis_index("s")
    wid = s * NC + c

    @pl.when(s < NS - 1)
    def _():
        pltpu.sync_copy(zeros_hbm, acc_sh.at[pl.ds(s * RPT, RPT)])

    @pl.when(s == NS - 1)
    def _():
        pltpu.sync_copy(zeros_hbm.at[pl.ds(0, RPT_LAST)],
                        acc_sh.at[pl.ds((NS - 1) * RPT, RPT_LAST)])

    plsc.subcore_barrier()

    bufs = (rows0, rows1, rows2)
    gsems = (gs0, gs1, gs2)
    ssems = (ss0, ss1, ss2)

    base = wid * EPW
    pltpu.sync_copy(row_hbm.at[pl.ds(base, EPW)], ridx)
    pltpu.sync_copy(col_hbm.at[pl.ds(base, EPW)], cidx)
    pltpu.async_copy(hs_hbm.at[ridx.at[pl.ds(0, K)]], rows0, gs0)
    pltpu.async_copy(hs_hbm.at[ridx.at[pl.ds(K, K)]], rows1, gs1)

    def chunk(i, t):
        buf = bufs[t]
        tn = (t + 2) % 3
        pltpu.make_async_copy(
            hs_hbm.at[ridx.at[pl.ds(i * K, K)]], buf, gsems[t]).wait()
        pltpu.async_copy(
            buf, acc_sh.at[cidx.at[pl.ds(i * K, K)]], ssems[t], add=True)

        @pl.when(jnp.logical_and(i + 2 < CPW, i >= 1))
        def _():
            pltpu.make_async_copy(
                bufs[tn], acc_sh.at[cidx.at[pl.ds(0, K)]], ssems[tn]).wait()

        @pl.when(i + 2 < CPW)
        def _():
            pltpu.async_copy(
                hs_hbm.at[ridx.at[pl.ds((i + 2) * K, K)]], bufs[tn], gsems[tn])

    def triple(j, _):
        for b in range(3):
            chunk(3 * j + b, b)
        return _

    lax.fori_loop(0, CPW // 3, triple, None)
    for i in range(3 * (CPW // 3), CPW):
        chunk(i, i % 3)
    for i in range(CPW - 3, CPW):
        pltpu.make_async_copy(
            bufs[i % 3], acc_sh.at[cidx.at[pl.ds(0, K)]], ssems[i % 3]).wait()
    plsc.subcore_barrier()

    @pl.when(s < NS - 1)
    def _():
        pltpu.sync_copy(acc_sh.at[pl.ds(s * RPT, RPT)],
                        out_hbm.at[pl.ds(c * NP + s * RPT, RPT)])

    @pl.when(s == NS - 1)
    def _():
        pltpu.sync_copy(acc_sh.at[pl.ds((NS - 1) * RPT, RPT_LAST)],
                        out_hbm.at[pl.ds(c * NP + (NS - 1) * RPT, RPT_LAST)])


_edge_kernel = functools.partial(
    pl.kernel,
    out_type=jax.ShapeDtypeStruct((NC * NP, D), jnp.float32),
    mesh=_mesh(),
    scratch_types=[
        pltpu.VMEM((EPW,), jnp.int32),
        pltpu.VMEM((EPW,), jnp.int32),
        pltpu.VMEM((K, D), jnp.float32),
        pltpu.VMEM((K, D), jnp.float32),
        pltpu.VMEM((K, D), jnp.float32),
        pltpu.VMEM_SHARED((NP, D), jnp.float32),
        pltpu.SemaphoreType.DMA,
        pltpu.SemaphoreType.DMA,
        pltpu.SemaphoreType.DMA,
        pltpu.SemaphoreType.DMA,
        pltpu.SemaphoreType.DMA,
        pltpu.SemaphoreType.DMA,
    ],
)(_edge_body)


BR = 2000


def _linear_body(x_ref, w_ref, degp_ref, hs_ref, dinv_ref):
    deg = jnp.sum(degp_ref[...], axis=1, keepdims=True) + 1.0
    dinv = lax.rsqrt(deg)
    h = jnp.dot(x_ref[...], w_ref[...], preferred_element_type=jnp.float32)
    hs_ref[...] = h * dinv
    dinv_ref[...] = dinv


def _linear_tc(x, w, degp):
    return pl.pallas_call(
        _linear_body,
        grid=(N // BR,),
        in_specs=[
            pl.BlockSpec((BR, D), lambda i: (i, 0)),
            pl.BlockSpec((D, D), lambda i: (0, 0)),
            pl.BlockSpec((BR, NW), lambda i: (i, 0)),
        ],
        out_specs=[
            pl.BlockSpec((BR, D), lambda i: (i, 0)),
            pl.BlockSpec((BR, 1), lambda i: (i, 0)),
        ],
        out_shape=[
            jax.ShapeDtypeStruct((N, D), jnp.float32),
            jax.ShapeDtypeStruct((N, 1), jnp.float32),
        ],
    )(x, w, degp)


def _finish_body(accp_ref, hs_ref, dinv_ref, b_ref, out_ref):
    acc = accp_ref[0] + accp_ref[1] + hs_ref[...]
    out_ref[...] = jnp.maximum(acc * dinv_ref[...] + b_ref[...], 0.0)


def _finish_tc(accp, hs, dinv, b2):
    return pl.pallas_call(
        _finish_body,
        grid=(N // BR,),
        in_specs=[
            pl.BlockSpec((NC, BR, D), lambda i: (0, i, 0)),
            pl.BlockSpec((BR, D), lambda i: (i, 0)),
            pl.BlockSpec((BR, 1), lambda i: (i, 0)),
            pl.BlockSpec((1, D), lambda i: (0, 0)),
        ],
        out_specs=pl.BlockSpec((BR, D), lambda i: (i, 0)),
        out_shape=jax.ShapeDtypeStruct((N, D), jnp.float32),
    )(accp, hs, dinv, b2)


@jax.jit
def _run(A, X, W, b):
    A = A.astype(jnp.int32)
    col_p = A[1]
    row_p = lax.optimization_barrier(A)[0]

    zerosD = jnp.zeros((RPT, D), jnp.float32)

    deg_flat = _deg_kernel(col_p)
    degp = deg_flat.reshape(NW, NP).T[:N]

    hs, dinv = _linear_tc(X, W, degp)

    acc_flat = _edge_kernel(hs, row_p, col_p, zerosD)
    accp = acc_flat.reshape(NC, NP, D)

    return _finish_tc(accp, hs, dinv, b.reshape(1, D))


def kernel(A, X, W, b):
    return _run(A, X, W, b)


# speedup vs baseline: 1.0008x; 1.0008x over previous
import functools

import jax
import jax.numpy as jnp
from jax import lax
from jax.experimental import pallas as pl
from jax.experimental.pallas import tpu as pltpu
from jax.experimental.pallas import tpu_sc as plsc

N = 10000
D = 128
E = 320000

NC = 2
NS = 16
NW = NC * NS

K = 80
CPW = 125
EPW = CPW * K
NP = 10000
RPT = 632
RPT_LAST = NP - 15 * RPT

_mesh = lambda: plsc.VectorSubcoreMesh(core_axis_name="c", subcore_axis_name="s")


def _deg_body(col_hbm, out_hbm, cidx, hist):
    c = lax.axis_index("c")
    s = lax.axis_index("s")
    wid = s * NC + c
    pltpu.sync_copy(col_hbm.at[pl.ds(wid * EPW, EPW)], cidx)
    zeros16 = jnp.zeros((16,), jnp.float32)

    def zstep(i, _):
        hist[pl.ds(i * 16, 16)] = zeros16
        return _

    lax.fori_loop(0, NP // 16, zstep, None)
    ones16 = jnp.ones((16,), jnp.float32)

    def step(j, _):
        ids = cidx[pl.ds(j * 16, 16)]
        plsc.addupdate_scatter(hist, [ids], ones16)
        return _

    lax.fori_loop(0, EPW // 16, step, None)
    pltpu.sync_copy(hist, out_hbm.at[pl.ds(wid * NP, NP)])


_deg_kernel = functools.partial(
    pl.kernel,
    out_type=jax.ShapeDtypeStruct((NW * NP,), jnp.float32),
    mesh=_mesh(),
    scratch_types=[
        pltpu.VMEM((EPW,), jnp.int32),
        pltpu.VMEM((NP,), jnp.float32),
    ],
    compiler_params=pltpu.CompilerParams(needs_layout_passes=False),
)(_deg_body)


def _edge_body(hs_hbm, row_hbm, col_hbm, zeros_hbm, out_hbm,
               ridx, cidx, rows0, rows1, rows2, acc_sh,
               gs0, gs1, gs2, ss0, ss1, ss2):
    c = lax.axis_index("c")
    s = lax.axis_index("s")
    wid = s * NC + c

    @pl.when(s < NS - 1)
    def _():
        pltpu.sync_copy(zeros_hbm, acc_sh.at[pl.ds(s * RPT, RPT)])

    @pl.when(s == NS - 1)
    def _():
        pltpu.sync_copy(zeros_hbm.at[pl.ds(0, RPT_LAST)],
                        acc_sh.at[pl.ds((NS - 1) * RPT, RPT_LAST)])

    plsc.subcore_barrier()

    bufs = (rows0, rows1, rows2)
    gsems = (gs0, gs1, gs2)
    ssems = (ss0, ss1, ss2)

    base = wid * EPW
    pltpu.sync_copy(row_hbm.at[pl.ds(base, EPW)], ridx)
    pltpu.sync_copy(col_hbm.at[pl.ds(base, EPW)], cidx)
    pltpu.async_copy(hs_hbm.at[ridx.at[pl.ds(0, K)]], rows0, gs0)
    pltpu.async_copy(hs_hbm.at[ridx.at[pl.ds(K, K)]], rows1, gs1)

    def chunk(i, t):
        buf = bufs[t]
        tn = (t + 2) % 3
        pltpu.make_async_copy(
            hs_hbm.at[ridx.at[pl.ds(i * K, K)]], buf, gsems[t]).wait()
        pltpu.async_copy(
            buf, acc_sh.at[cidx.at[pl.ds(i * K, K)]], ssems[t], add=True)

        @pl.when(jnp.logical_and(i + 2 < CPW, i >= 1))
        def _():
            pltpu.make_async_copy(
                bufs[tn], acc_sh.at[cidx.at[pl.ds(0, K)]], ssems[tn]).wait()

        @pl.when(i + 2 < CPW)
        def _():
            pltpu.async_copy(
                hs_hbm.at[ridx.at[pl.ds((i + 2) * K, K)]], bufs[tn], gsems[tn])

    def triple(j, _):
        for b in range(3):
            chunk(3 * j + b, b)
        return _

    lax.fori_loop(0, CPW // 3, triple, None)
    for i in range(3 * (CPW // 3), CPW):
        chunk(i, i % 3)
    for i in range(CPW - 3, CPW):
        pltpu.make_async_copy(
            bufs[i % 3], acc_sh.at[cidx.at[pl.ds(0, K)]], ssems[i % 3]).wait()
    plsc.subcore_barrier()

    @pl.when(s < NS - 1)
    def _():
        pltpu.sync_copy(acc_sh.at[pl.ds(s * RPT, RPT)],
                        out_hbm.at[pl.ds(c * NP + s * RPT, RPT)])

    @pl.when(s == NS - 1)
    def _():
        pltpu.sync_copy(acc_sh.at[pl.ds((NS - 1) * RPT, RPT_LAST)],
                        out_hbm.at[pl.ds(c * NP + (NS - 1) * RPT, RPT_LAST)])


_edge_kernel = functools.partial(
    pl.kernel,
    out_type=jax.ShapeDtypeStruct((NC * NP, D), jnp.float32),
    mesh=_mesh(),
    scratch_types=[
        pltpu.VMEM((EPW,), jnp.int32),
        pltpu.VMEM((EPW,), jnp.int32),
        pltpu.VMEM((K, D), jnp.float32),
        pltpu.VMEM((K, D), jnp.float32),
        pltpu.VMEM((K, D), jnp.float32),
        pltpu.VMEM_SHARED((NP, D), jnp.float32),
        pltpu.SemaphoreType.DMA,
        pltpu.SemaphoreType.DMA,
        pltpu.SemaphoreType.DMA,
        pltpu.SemaphoreType.DMA,
        pltpu.SemaphoreType.DMA,
        pltpu.SemaphoreType.DMA,
    ],
)(_edge_body)


BR = 2000


def _linear_body(x_ref, w_ref, degp_ref, hs_ref, dinv_ref):
    deg = jnp.sum(degp_ref[...], axis=1, keepdims=True) + 1.0
    dinv = lax.rsqrt(deg)
    h = jnp.dot(x_ref[...], w_ref[...], preferred_element_type=jnp.float32)
    hs_ref[...] = h * dinv
    dinv_ref[...] = dinv


def _linear_tc(x, w, degp):
    return pl.pallas_call(
        _linear_body,
        grid=(N // BR,),
        in_specs=[
            pl.BlockSpec((BR, D), lambda i: (i, 0)),
            pl.BlockSpec((D, D), lambda i: (0, 0)),
            pl.BlockSpec((BR, NW), lambda i: (i, 0)),
        ],
        out_specs=[
            pl.BlockSpec((BR, D), lambda i: (i, 0)),
            pl.BlockSpec((BR, 1), lambda i: (i, 0)),
        ],
        out_shape=[
            jax.ShapeDtypeStruct((N, D), jnp.float32),
            jax.ShapeDtypeStruct((N, 1), jnp.float32),
        ],
    )(x, w, degp)


def _finish_body(accp_ref, hs_ref, dinv_ref, b_ref, out_ref):
    acc = accp_ref[0] + accp_ref[1] + hs_ref[...]
    out_ref[...] = jnp.maximum(acc * dinv_ref[...] + b_ref[...], 0.0)


def _finish_tc(accp, hs, dinv, b2):
    return pl.pallas_call(
        _finish_body,
        grid=(N // BR,),
        in_specs=[
            pl.BlockSpec((NC, BR, D), lambda i: (0, i, 0)),
            pl.BlockSpec((BR, D), lambda i: (i, 0)),
            pl.BlockSpec((BR, 1), lambda i: (i, 0)),
            pl.BlockSpec((1, D), lambda i: (0, 0)),
        ],
        out_specs=pl.BlockSpec((BR, D), lambda i: (i, 0)),
        out_shape=jax.ShapeDtypeStruct((N, D), jnp.float32),
    )(accp, hs, dinv, b2)


@jax.jit
def _run(A, X, W, b):
    A = A.astype(jnp.int32)
    row_p = A[0]
    col_p = A[1]

    zerosD = jnp.zeros((RPT, D), jnp.float32)

    deg_flat = _deg_kernel(col_p)
    degp = deg_flat.reshape(NW, NP).T[:N]

    hs, dinv = _linear_tc(X, W, degp)

    acc_flat = _edge_kernel(hs, row_p, col_p, zerosD)
    accp = acc_flat.reshape(NC, NP, D)

    return _finish_tc(accp, hs, dinv, b.reshape(1, D))


def kernel(A, X, W, b):
    return _run(A, X, W, b)
